# baseline (speedup 1.0000x reference)
import functools

import jax
import jax.numpy as jnp
from jax import lax
from jax.experimental import pallas as pl
from jax.experimental.pallas import tpu as pltpu
from jax.experimental.pallas import tpu_sc as plsc

L = 12
SRC = 2048 + 2
TGT = 2048 + 2
D = 1024

_TBL = L * SRC * D
_NW = 32
_PW = _TBL // _NW
_NCH = 30
_CH = _PW // _NCH
_NBUF = 4
_TOTAL_CHUNKS = 3 * _NCH


def _sc_copy(enc_hbm, self_hbm, cross_hbm, out_hbm, *scratch):
    nc = plsc.get_sparse_core_info().num_cores
    wid = lax.axis_index("s") * nc + lax.axis_index("c")
    base = wid * _PW
    srcs = (enc_hbm, self_hbm, cross_hbm)
    bufs = scratch[:_NBUF]
    rsems = scratch[_NBUF:2 * _NBUF]
    wsems = scratch[2 * _NBUF:]

    def rd(k):
        t, c = divmod(k, _NCH)
        b = k % _NBUF
        src = srcs[t].at[pl.ds(base + c * _CH, _CH)]
        return pltpu.make_async_copy(src, bufs[b], rsems[b])

    def wr(k):
        t, c = divmod(k, _NCH)
        b = k % _NBUF
        dst = out_hbm.at[pl.ds(t * _TBL + base + c * _CH, _CH)]
        return pltpu.make_async_copy(bufs[b], dst, wsems[b])

    rd(0).start()
    for k in range(_TOTAL_CHUNKS):
        if k + 1 < _TOTAL_CHUNKS:
            if k + 1 >= _NBUF:
                wr(k + 1 - _NBUF).wait()
            rd(k + 1).start()
        rd(k).wait()
        wr(k).start()
    for j in range(_TOTAL_CHUNKS - _NBUF, _TOTAL_CHUNKS):
        wr(j).wait()


def kernel(bsz, enc_w, self_w, cross_w):
    del bsz
    enc2 = enc_w.reshape(_TBL)
    self2 = self_w.reshape(_TBL)
    cross2 = cross_w.reshape(_TBL)
    mesh = plsc.VectorSubcoreMesh(core_axis_name="c", subcore_axis_name="s")
    run = pl.kernel(
        _sc_copy,
        out_type=jax.ShapeDtypeStruct((3 * _TBL,), jnp.float32),
        mesh=mesh,
        scratch_types=(
            [pltpu.VMEM((_CH,), jnp.float32)] * _NBUF
            + [pltpu.SemaphoreType.DMA] * (2 * _NBUF)
        ),
    )
    out = run(enc2, self2, cross2)
    return out.reshape(3, L, SRC, D)


# speedup vs baseline: 1.1663x; 1.1663x over previous
import jax
import jax.numpy as jnp
from jax.experimental import pallas as pl
from jax.experimental.pallas import tpu as pltpu

L = 12
SRC = 2048 + 2
TGT = 2048 + 2
D = 1024

_ROWS = L * SRC
_RB = 600
_NCH = _ROWS // _RB
_TOTAL = 3 * _NCH
_NBUF = 8


def _dma_pipeline(enc, selfw, cross, out, buf, rsem, wsem):
    srcs = (enc, selfw, cross)

    def rd(k):
        t, c = divmod(k, _NCH)
        b = k % _NBUF
        src = srcs[t].at[pl.ds(c * _RB, _RB), :]
        return pltpu.make_async_copy(src, buf.at[b], rsem.at[b])

    def wr(k):
        t, c = divmod(k, _NCH)
        b = k % _NBUF
        dst = out.at[pl.ds(t * _ROWS + c * _RB, _RB), :]
        return pltpu.make_async_copy(buf.at[b], dst, wsem.at[b])

    rd(0).start()
    for k in range(_TOTAL):
        if k + 1 < _TOTAL:
            if k + 1 >= _NBUF:
                wr(k + 1 - _NBUF).wait()
            rd(k + 1).start()
        rd(k).wait()
        wr(k).start()
    for j in range(_TOTAL - _NBUF, _TOTAL):
        wr(j).wait()


def kernel(bsz, enc_w, self_w, cross_w):
    del bsz
    enc2 = enc_w.reshape(_ROWS, D)
    self2 = self_w.reshape(_ROWS, D)
    cross2 = cross_w.reshape(_ROWS, D)
    out = pl.pallas_call(
        _dma_pipeline,
        in_specs=[
            pl.BlockSpec(memory_space=pl.ANY),
            pl.BlockSpec(memory_space=pl.ANY),
            pl.BlockSpec(memory_space=pl.ANY),
        ],
        out_specs=pl.BlockSpec(memory_space=pl.ANY),
        out_shape=jax.ShapeDtypeStruct((3 * _ROWS, D), jnp.float32),
        scratch_shapes=[
            pltpu.VMEM((_NBUF, _RB, D), jnp.float32),
            pltpu.SemaphoreType.DMA((_NBUF,)),
            pltpu.SemaphoreType.DMA((_NBUF,)),
        ],
    )(enc2, self2, cross2)
    return out.reshape(3, L, SRC, D)
